# baseline (speedup 1.0000x reference)
import functools

import jax
import jax.numpy as jnp
from jax import lax
from jax.experimental import pallas as pl
from jax.experimental.pallas import tpu as pltpu
from jax.experimental.pallas import tpu_sc as plsc

_B = 500
_NF = 20
_N = _B * _NF
_E = 160000
_FD = 64
_HID = 128
_H1 = 4

_NC = 2
_NS = 16
_L = 16
_EPT = _E // _NS
_CH = 80
_NCH = _EPT // _CH
_NCHT = 128
_EPTP = _NCHT * _CH
_NP = 10240
_RPT = _NP // _NS
_ZC = 64
_NZ = _RPT // _ZC
_TN = 1000

_SC_PARAMS = pltpu.CompilerParams(needs_layout_passes=False,
                                  use_tc_tiling_on_sc=False)


def _leaky(x):
    return jnp.where(x >= 0.0, x, 0.2 * x)


def _stage_a(fe, w_align, b_align, ln_g, ln_b, logits_rows, w1, a_src1, a_dst1):
    grid = (_N // _TN,)

    def body(fe_ref, wa_ref, ba_ref, lg_ref, lb_ref, ml_ref, w1_ref, s_ref, d_ref,
             h1_ref, as_ref, ad_ref, gate_ref):
        x = jnp.dot(fe_ref[...], wa_ref[...], preferred_element_type=jnp.float32)
        x = x + ba_ref[...]
        mu = jnp.mean(x, axis=-1, keepdims=True)
        xc = x - mu
        var = jnp.mean(xc * xc, axis=-1, keepdims=True)
        xn = xc * lax.rsqrt(var + 1e-5) * lg_ref[...] + lb_ref[...]
        gate = jax.nn.sigmoid(ml_ref[...])
        xg = xn * gate
        h1 = jnp.dot(xg, w1_ref[...], preferred_element_type=jnp.float32)
        h1_ref[...] = h1
        hr = h1.reshape(_TN, _H1, _HID)
        as_ref[...] = jnp.sum(hr * s_ref[...][None], axis=-1)
        ad_ref[...] = jnp.sum(hr * d_ref[...][None], axis=-1)
        gate_ref[...] = gate

    full = lambda a: pl.BlockSpec(a.shape, lambda i: (0,) * a.ndim)
    return pl.pallas_call(
        body,
        grid=grid,
        in_specs=[
            pl.BlockSpec((_TN, _FD), lambda i: (i, 0)),
            full(w_align), full(b_align), full(ln_g), full(ln_b),
            pl.BlockSpec((_TN, 1), lambda i: (i, 0)),
            full(w1), full(a_src1), full(a_dst1),
        ],
        out_specs=[
            pl.BlockSpec((_TN, _H1 * _HID), lambda i: (i, 0)),
            pl.BlockSpec((_TN, _H1), lambda i: (i, 0)),
            pl.BlockSpec((_TN, _H1), lambda i: (i, 0)),
            pl.BlockSpec((_TN, 1), lambda i: (i, 0)),
        ],
        out_shape=[
            jax.ShapeDtypeStruct((_N, _H1 * _HID), jnp.float32),
            jax.ShapeDtypeStruct((_N, _H1), jnp.float32),
            jax.ShapeDtypeStruct((_N, _H1), jnp.float32),
            jax.ShapeDtypeStruct((_N, 1), jnp.float32),
        ],
    )(fe, w_align, b_align, ln_g, ln_b, logits_rows, w1, a_src1, a_dst1)


def _gat_conv_sc(hv, ast, adt, src, dst, zrows, mult, cmul, npass):
    W = 64
    mesh = plsc.VectorSubcoreMesh(core_axis_name="c", subcore_axis_name="s",
                                  num_cores=_NC, num_subcores=_NS)
    out_type = (
        jax.ShapeDtypeStruct((_NC, npass, _NP, W), jnp.float32),
        jax.ShapeDtypeStruct((_NC, _NP, 8), jnp.float32),
    )
    scratch = [
        pltpu.VMEM((_EPTP,), jnp.int32),
        pltpu.VMEM((_EPTP,), jnp.int32),
        pltpu.VMEM((_N,), jnp.float32),
        pltpu.VMEM((_N,), jnp.float32),
    ] + [pltpu.VMEM((_CH,), jnp.int32)] * 4 \
      + [pltpu.VMEM((_CH,), jnp.int32)] * 4 \
      + [pltpu.VMEM((_CH,), jnp.float32)] * 4 \
      + [pltpu.VMEM((_CH, 8), jnp.float32)] * 4 \
      + [pltpu.VMEM((_CH, W), jnp.float32)] * 4 + [
        pltpu.VMEM((_ZC, W), jnp.float32),
        pltpu.VMEM((_ZC, 8), jnp.float32),
        pltpu.VMEM((_ZC, W), jnp.float32),
        pltpu.VMEM((_ZC, 8), jnp.float32),
        pltpu.VMEM_SHARED((_NP, W), jnp.float32),
        pltpu.VMEM_SHARED((_NP, 8), jnp.float32),
    ] + [pltpu.SemaphoreType.DMA] * 12

    @functools.partial(pl.kernel, out_type=out_type, mesh=mesh,
                       compiler_params=_SC_PARAMS, scratch_types=scratch)
    def k(hv_h, as_h, ad_h, src_h, dst_h, zr_h, acc_o, den_o,
          src_l, dst_l, as_tr, ad_tr,
          i0, i1, i2, i3, t0, t1, t2, t3, e0, e1, e2, e3,
          n0, n1, n2, n3, r0, r1, r2, r3,
          zbuf, zbuf2, fbuf, fbuf2, accs, dens,
          g0, g1, g2, g3, s0, s1, s2, s3, d0, d1, d2, d3):
        c = lax.axis_index("c")
        s = lax.axis_index("s")
        ebase = s * _EPT
        rbase = s * _RPT
        pltpu.sync_copy(src_h.at[pl.ds(ebase, _EPT)],
                        src_l.at[pl.ds(0, _EPT)])
        pltpu.sync_copy(dst_h.at[pl.ds(ebase, _EPT)],
                        dst_l.at[pl.ds(0, _EPT)])

        zv = jnp.zeros((_L,), jnp.float32)
        zvi = jnp.zeros((_L,), jnp.int32)
        iota16 = lax.iota(jnp.int32, _L)

        for j in range((_EPTP - _EPT) // _L):
            src_l[pl.ds(_EPT + j * _L, _L)] = zvi
            dst_l[pl.ds(_EPT + j * _L, _L)] = zvi

        def _zb(i, carry):
            for f in range(W // _L):
                zbuf[i, pl.ds(f * _L, _L)] = zv
            return carry
        lax.fori_loop(0, _ZC, _zb, 0)
        pltpu.sync_copy(zr_h.at[pl.ds(0, _ZC)], zbuf2)

        def _zdr():
            for nx in (n0, n1, n2, n3):
                pltpu.sync_copy(zr_h, nx)

        for z in range(_NZ):
            pltpu.sync_copy(zbuf, accs.at[pl.ds(rbase + z * _ZC, _ZC)])
            pltpu.sync_copy(zbuf2, dens.at[pl.ds(rbase + z * _ZC, _ZC)])
        plsc.subcore_barrier()

        ept_s = jnp.full((_L,), _EPT, jnp.int32)

        def pass_body(o_p, carry):
            tid = o_p // 2
            gq = c * cmul + o_p
            den_gate = jnp.where(o_p % 2 == 0, 1.0, 0.0)
            dcol = jnp.full((_L,), 1, jnp.int32) * tid
            pltpu.sync_copy(as_h.at[c, tid], as_tr)
            pltpu.sync_copy(ad_h.at[c, tid], ad_tr)
            _zdr()

            def score(ch, idx_x, dst_x, exb_x, denr_x):
                off = ch * _CH
                for j in range(_CH // _L):
                    o = pl.ds(off + j * _L, _L)
                    sj = src_l[o]
                    dj = dst_l[o]
                    av = plsc.load_gather(as_tr, [sj])
                    bv = plsc.load_gather(ad_tr, [dj])
                    ex = jnp.exp(_leaky(av + bv))
                    m = (iota16 + (off + j * _L)) < ept_s
                    ex = jnp.where(m, ex, 0.0)
                    oc = pl.ds(j * _L, _L)
                    exb_x[oc] = ex
                    idx_x[oc] = sj * mult + gq
                    dst_x[oc] = dj
                    plsc.store_scatter(denr_x, [iota16 + j * _L, dcol],
                                       ex * den_gate)

            bufs = ((i0, t0, e0, n0, r0, g0, s0, d0),
                    (i1, t1, e1, n1, r1, g1, s1, d1),
                    (i2, t2, e2, n2, r2, g2, s2, d2),
                    (i3, t3, e3, n3, r3, g3, s3, d3))

            def issue(ch, buf):
                idx_x, dst_x, exb_x, denr_x, rows_x, g_x, s_x, d_x = buf
                score(ch, idx_x, dst_x, exb_x, denr_x)
                pltpu.async_copy(hv_h.at[idx_x], rows_x, g_x)

            def ret(buf):
                idx_x, dst_x, exb_x, denr_x, rows_x, g_x, s_x, d_x = buf
                pltpu.make_async_copy(hv_h.at[idx_x], rows_x, g_x).wait()

                def _scale(qq, cc):
                    exv = exb_x[pl.ds(qq * _L, _L)]
                    for lane in range(_L):
                        i = qq * _L + lane
                        exs = exv[lane]
                        for f in range(W // _L):
                            fo = pl.ds(f * _L, _L)
                            rows_x[i, fo] = rows_x[i, fo] * exs
                    return cc
                lax.fori_loop(0, _CH // _L, _scale, 0)
                pltpu.async_copy(rows_x, accs.at[dst_x], s_x, add=True)
                pltpu.async_copy(denr_x, dens.at[dst_x], d_x, add=True)

            def wts(buf):
                idx_x, dst_x, exb_x, denr_x, rows_x, g_x, s_x, d_x = buf
                pltpu.make_async_copy(rows_x, accs.at[dst_x], s_x).wait()
                pltpu.make_async_copy(denr_x, dens.at[dst_x], d_x).wait()

            issue(0, bufs[0])
            issue(1, bufs[1])
            issue(2, bufs[2])
            ret(bufs[0])
            issue(3, bufs[3])
            ret(bufs[1])

            def _quad(t, cc):
                for i in range(4):
                    wts(bufs[i])
                    issue(t * 4 + i, bufs[i])
                    ret(bufs[(i + 2) % 4])
                return cc
            lax.fori_loop(1, _NCHT // 4, _quad, 0)
            ret(bufs[2])
            ret(bufs[3])
            for i in range(4):
                wts(bufs[i])

            plsc.subcore_barrier()
            for z in range(_NZ):
                rz = pl.ds(rbase + z * _ZC, _ZC)
                pltpu.sync_copy(accs.at[rz], fbuf)
                pltpu.sync_copy(fbuf, acc_o.at[c, o_p, rz])

            @pl.when(o_p < npass - 1)
            def _rezero():
                for z in range(_NZ):
                    rz = pl.ds(rbase + z * _ZC, _ZC)
                    pltpu.sync_copy(zbuf, accs.at[rz])
            plsc.subcore_barrier()
            return carry
        lax.fori_loop(0, npass, pass_body, 0)

        for z in range(_NZ):
            rz = pl.ds(rbase + z * _ZC, _ZC)
            pltpu.sync_copy(dens.at[rz], fbuf2)
            pltpu.sync_copy(fbuf2, den_o.at[c, rz])

    return k(hv, ast, adt, src, dst, zrows)


def _stage_c(acc1, den1, h1, as1, ad1, w2, b1r, a_s2, a_d2):
    grid = (_N // _TN,)

    def body(acc_ref, den_ref, h1_ref, s1_ref, d1_ref, w2_ref, b1_ref,
             s2w_ref, d2w_ref, h2_ref, as2_ref, ad2_ref):
        acc = acc_ref[...]
        den = den_ref[...]
        h1 = h1_ref[...]
        exs = jnp.exp(_leaky(s1_ref[...] + d1_ref[...]))
        den4 = jnp.concatenate([den[0, :, 0:2], den[1, :, 0:2]], axis=-1)
        den4 = den4 + exs + 1e-16
        outs = []
        for g in range(_H1):
            ung = jnp.concatenate([acc[g, 0], acc[g, 1]], axis=-1)
            hg = h1[:, g * _HID:(g + 1) * _HID]
            ung = ung + exs[:, g:g + 1] * hg
            og = ung / den4[:, g:g + 1] + b1_ref[0, g * _HID:(g + 1) * _HID][None]
            outs.append(og)
        x2 = jnp.concatenate(outs, axis=-1)
        x2 = jnp.where(x2 > 0.0, x2, jnp.exp(x2) - 1.0)
        h2 = jnp.dot(x2, w2_ref[...], preferred_element_type=jnp.float32)
        h2_ref[...] = h2
        as2_ref[...] = jnp.sum(h2 * s2w_ref[...], axis=-1, keepdims=True)
        ad2_ref[...] = jnp.sum(h2 * d2w_ref[...], axis=-1, keepdims=True)

    full = lambda a: pl.BlockSpec(a.shape, lambda i: (0,) * a.ndim)
    return pl.pallas_call(
        body,
        grid=grid,
        in_specs=[
            pl.BlockSpec((_H1, 2, _TN, 64), lambda i: (0, 0, i, 0)),
            pl.BlockSpec((_NC, _TN, 8), lambda i: (0, i, 0)),
            pl.BlockSpec((_TN, _H1 * _HID), lambda i: (i, 0)),
            pl.BlockSpec((_TN, _H1), lambda i: (i, 0)),
            pl.BlockSpec((_TN, _H1), lambda i: (i, 0)),
            full(w2), full(b1r), full(a_s2), full(a_d2),
        ],
        out_specs=[
            pl.BlockSpec((_TN, _HID), lambda i: (i, 0)),
            pl.BlockSpec((_TN, 1), lambda i: (i, 0)),
            pl.BlockSpec((_TN, 1), lambda i: (i, 0)),
        ],
        out_shape=[
            jax.ShapeDtypeStruct((_N, _HID), jnp.float32),
            jax.ShapeDtypeStruct((_N, 1), jnp.float32),
            jax.ShapeDtypeStruct((_N, 1), jnp.float32),
        ],
    )(acc1, den1, h1, as1, ad1, w2, b1r, a_s2, a_d2)


def _stage_e(acc2, den2, h2, as2, ad2, b2r):
    def body(acc_ref, den_ref, h2_ref, s2_ref, d2_ref, b2_ref, out_ref):
        a2 = acc_ref[...]
        h2 = h2_ref[...]
        exs = jnp.exp(_leaky(s2_ref[...] + d2_ref[...]))
        un = jnp.concatenate([a2[0, :_N], a2[1, :_N]], axis=-1)
        un = un + exs * h2
        den = den_ref[...][0, :_N, 0:1] + exs + 1e-16
        o = un / den + b2_ref[...]
        out_ref[...] = jnp.mean(o.reshape(_B, _NF, _HID), axis=1)

    full = lambda a: pl.BlockSpec(a.shape, lambda: (0,) * a.ndim)
    return pl.pallas_call(
        body,
        in_specs=[
            full(acc2), full(den2), full(h2), full(as2), full(ad2), full(b2r),
        ],
        out_specs=[pl.BlockSpec((_B, _HID), lambda: (0, 0))],
        out_shape=[jax.ShapeDtypeStruct((_B, _HID), jnp.float32)],
    )(acc2, den2, h2, as2, ad2, b2r)[0]


def kernel(field_emb, W_align, b_align, ln_g, ln_b, mask_logits,
           W1, a_src1, a_dst1, b1, W2, a_src2, a_dst2, b2,
           edge_index, batch_idx):
    fe = field_emb.reshape(_N, _FD)
    logits_rows = jnp.tile(mask_logits, _B)[:, None]
    h1, as1, ad1, gate_rows = _stage_a(
        fe, W_align, b_align.reshape(1, -1), ln_g.reshape(1, -1),
        ln_b.reshape(1, -1), logits_rows, W1, a_src1, a_dst1)
    src = edge_index[0]
    dst = edge_index[1]
    zrows = jnp.zeros((_CH, 8), jnp.float32)
    acc1, den1 = _gat_conv_sc(h1.reshape(_N * 8, 64),
                              as1.T.reshape(_NC, 2, _N),
                              ad1.T.reshape(_NC, 2, _N), src, dst, zrows,
                              8, 4, 4)
    acc1 = acc1.reshape(_H1, 2, _NP, 64)
    h2, as2, ad2 = _stage_c(acc1, den1, h1, as1, ad1, W2,
                            b1.reshape(1, -1), a_src2, a_dst2)
    as2b = jnp.broadcast_to(as2.reshape(1, 1, _N), (_NC, 2, _N))
    ad2b = jnp.broadcast_to(ad2.reshape(1, 1, _N), (_NC, 2, _N))
    acc2, den2 = _gat_conv_sc(h2.reshape(_N * 2, 64),
                              as2b, ad2b, src, dst, zrows,
                              2, 1, 1)
    acc2 = acc2.reshape(_NC, _NP, 64)
    ge = _stage_e(acc2, den2, h2, as2, ad2, b2.reshape(1, -1))
    gate = gate_rows[:_NF, 0]
    return (ge, gate)


# speedup vs baseline: 1.1224x; 1.1224x over previous
import functools

import jax
import jax.numpy as jnp
from jax import lax
from jax.experimental import pallas as pl
from jax.experimental.pallas import tpu as pltpu
from jax.experimental.pallas import tpu_sc as plsc

_B = 500
_NF = 20
_N = _B * _NF
_E = 160000
_FD = 64
_HID = 128
_H1 = 4

_NC = 2
_NS = 16
_L = 16
_EPT = _E // _NS
_CH = 80
_NCH = _EPT // _CH
_NCHT = 128
_EPTP = _NCHT * _CH
_NP = 10240
_RPT = _NP // _NS
_ZC = 64
_NZ = _RPT // _ZC
_TN = 1000

_SC_PARAMS = pltpu.CompilerParams(needs_layout_passes=False,
                                  use_tc_tiling_on_sc=False)


def _leaky(x):
    return jnp.where(x >= 0.0, x, 0.2 * x)


def _stage_a(fe, w_align, b_align, ln_g, ln_b, logits_rows, w1, a_src1, a_dst1):
    grid = (_N // _TN,)

    def body(fe_ref, wa_ref, ba_ref, lg_ref, lb_ref, ml_ref, w1_ref, s_ref, d_ref,
             h1_ref, as_ref, ad_ref, gate_ref):
        x = jnp.dot(fe_ref[...], wa_ref[...], preferred_element_type=jnp.float32)
        x = x + ba_ref[...]
        mu = jnp.mean(x, axis=-1, keepdims=True)
        xc = x - mu
        var = jnp.mean(xc * xc, axis=-1, keepdims=True)
        xn = xc * lax.rsqrt(var + 1e-5) * lg_ref[...] + lb_ref[...]
        gate = jax.nn.sigmoid(ml_ref[...])
        xg = xn * gate
        h1 = jnp.dot(xg, w1_ref[...], preferred_element_type=jnp.float32)
        h1_ref[...] = h1
        hr = h1.reshape(_TN, _H1, _HID)
        as_ref[...] = jnp.sum(hr * s_ref[...][None], axis=-1)
        ad_ref[...] = jnp.sum(hr * d_ref[...][None], axis=-1)
        gate_ref[...] = gate

    full = lambda a: pl.BlockSpec(a.shape, lambda i: (0,) * a.ndim)
    return pl.pallas_call(
        body,
        grid=grid,
        in_specs=[
            pl.BlockSpec((_TN, _FD), lambda i: (i, 0)),
            full(w_align), full(b_align), full(ln_g), full(ln_b),
            pl.BlockSpec((_TN, 1), lambda i: (i, 0)),
            full(w1), full(a_src1), full(a_dst1),
        ],
        out_specs=[
            pl.BlockSpec((_TN, _H1 * _HID), lambda i: (i, 0)),
            pl.BlockSpec((_TN, _H1), lambda i: (i, 0)),
            pl.BlockSpec((_TN, _H1), lambda i: (i, 0)),
            pl.BlockSpec((_TN, 1), lambda i: (i, 0)),
        ],
        out_shape=[
            jax.ShapeDtypeStruct((_N, _H1 * _HID), jnp.float32),
            jax.ShapeDtypeStruct((_N, _H1), jnp.float32),
            jax.ShapeDtypeStruct((_N, _H1), jnp.float32),
            jax.ShapeDtypeStruct((_N, 1), jnp.float32),
        ],
    )(fe, w_align, b_align, ln_g, ln_b, logits_rows, w1, a_src1, a_dst1)


def _gat_conv_sc(hv, ast, adt, src, dst, zrows, mult, cmul, passes):
    W = 64
    npass = len(passes)
    mesh = plsc.VectorSubcoreMesh(core_axis_name="c", subcore_axis_name="s",
                                  num_cores=_NC, num_subcores=_NS)
    out_type = (
        jax.ShapeDtypeStruct((_NC, npass, _NP, W), jnp.float32),
        jax.ShapeDtypeStruct((_NC, _NP, 8), jnp.float32),
    )
    scratch = [
        pltpu.VMEM((_EPTP,), jnp.int32),
        pltpu.VMEM((_EPTP,), jnp.int32),
        pltpu.VMEM((_N,), jnp.float32),
        pltpu.VMEM((_N,), jnp.float32),
        pltpu.VMEM((_EPTP,), jnp.float32),
        pltpu.VMEM((_CH,), jnp.int32),
        pltpu.VMEM((_CH,), jnp.int32),
        pltpu.VMEM((_CH,), jnp.int32),
        pltpu.VMEM((_CH,), jnp.int32),
        pltpu.VMEM((_CH,), jnp.float32),
        pltpu.VMEM((_CH,), jnp.float32),
        pltpu.VMEM((_CH, 8), jnp.float32),
        pltpu.VMEM((_CH, 8), jnp.float32),
        pltpu.VMEM((_CH, W), jnp.float32),
        pltpu.VMEM((_CH, W), jnp.float32),
        pltpu.VMEM((_ZC, W), jnp.float32),
        pltpu.VMEM((_ZC, 8), jnp.float32),
        pltpu.VMEM((_ZC, W), jnp.float32),
        pltpu.VMEM((_ZC, 8), jnp.float32),
        pltpu.VMEM_SHARED((_NP, W), jnp.float32),
        pltpu.VMEM_SHARED((_NP, 8), jnp.float32),
        pltpu.SemaphoreType.DMA,
        pltpu.SemaphoreType.DMA,
    ]

    @functools.partial(pl.kernel, out_type=out_type, mesh=mesh,
                       compiler_params=_SC_PARAMS, scratch_types=scratch)
    def k(hv_h, as_h, ad_h, src_h, dst_h, zr_h, acc_o, den_o,
          src_l, dst_l, as_tr, ad_tr, exc,
          idx_a, idx_b, dst_a, dst_b, exb_a, exb_b, denr_a, denr_b,
          rows_a, rows_b, zbuf, zbuf2, fbuf, fbuf2, accs, dens, g_a, g_b):
        c = lax.axis_index("c")
        s = lax.axis_index("s")
        ebase = s * _EPT
        rbase = s * _RPT
        pltpu.sync_copy(src_h.at[pl.ds(ebase, _EPT)],
                        src_l.at[pl.ds(0, _EPT)])
        pltpu.sync_copy(dst_h.at[pl.ds(ebase, _EPT)],
                        dst_l.at[pl.ds(0, _EPT)])
        pltpu.sync_copy(as_h.at[c, passes[0][1]], as_tr)
        pltpu.sync_copy(ad_h.at[c, passes[0][1]], ad_tr)

        zv = jnp.zeros((_L,), jnp.float32)
        zvi = jnp.zeros((_L,), jnp.int32)
        iota16 = lax.iota(jnp.int32, _L)

        for j in range((_EPTP - _EPT) // _L):
            src_l[pl.ds(_EPT + j * _L, _L)] = zvi
            dst_l[pl.ds(_EPT + j * _L, _L)] = zvi
            exc[pl.ds(_EPT + j * _L, _L)] = zv

        def _zb(i, carry):
            for f in range(W // _L):
                zbuf[i, pl.ds(f * _L, _L)] = zv
            return carry
        lax.fori_loop(0, _ZC, _zb, 0)
        pltpu.sync_copy(zr_h.at[pl.ds(0, _ZC)], zbuf2)

        def _zdr():
            pltpu.sync_copy(zr_h, denr_a)
            pltpu.sync_copy(zr_h, denr_b)

        for z in range(_NZ):
            pltpu.sync_copy(zbuf, accs.at[pl.ds(rbase + z * _ZC, _ZC)])
            pltpu.sync_copy(zbuf2, dens.at[pl.ds(rbase + z * _ZC, _ZC)])
        plsc.subcore_barrier()

        ept_s = jnp.full((_L,), _EPT, jnp.int32)

        prev_tid = passes[0][1]
        for pi, (o_p, tid, den_on, den_col, from_cache) in enumerate(passes):
            gq = c * cmul + o_p
            if tid != prev_tid:
                pltpu.sync_copy(as_h.at[c, tid], as_tr)
                pltpu.sync_copy(ad_h.at[c, tid], ad_tr)
                prev_tid = tid
            if den_on:
                _zdr()

            def score(ch, idx_x, dst_x, exb_x, denr_x, den_on=den_on,
                      gq=gq, den_col=den_col, from_cache=from_cache):
                off = ch * _CH
                for j in range(_CH // _L):
                    o = pl.ds(off + j * _L, _L)
                    sj = src_l[o]
                    if from_cache:
                        ex = exc[o]
                    else:
                        dj = dst_l[o]
                        av = plsc.load_gather(as_tr, [sj])
                        bv = plsc.load_gather(ad_tr, [dj])
                        ex = jnp.exp(_leaky(av + bv))
                        m = (iota16 + (off + j * _L)) < ept_s
                        ex = jnp.where(m, ex, 0.0)
                        exc[o] = ex
                    oc = pl.ds(j * _L, _L)
                    exb_x[oc] = ex
                    idx_x[oc] = sj * mult + gq
                    dst_x[oc] = dst_l[o]
                    if den_on:
                        plsc.store_scatter(
                            denr_x,
                            [iota16 + j * _L,
                             jnp.full((_L,), den_col, jnp.int32)],
                            ex)

            def issue(ch, buf):
                idx_x, dst_x, exb_x, denr_x, rows_x, g_x = buf
                score(ch, idx_x, dst_x, exb_x, denr_x)
                pltpu.async_copy(hv_h.at[idx_x], rows_x, g_x)

            def ret(buf, den_on=den_on):
                idx_x, dst_x, exb_x, denr_x, rows_x, g_x = buf
                pltpu.make_async_copy(hv_h.at[idx_x], rows_x, g_x).wait()

                def _scale(qq, cc):
                    exv = exb_x[pl.ds(qq * _L, _L)]
                    for lane in range(_L):
                        i = qq * _L + lane
                        exs = exv[lane]
                        for f in range(W // _L):
                            fo = pl.ds(f * _L, _L)
                            rows_x[i, fo] = rows_x[i, fo] * exs
                    return cc
                lax.fori_loop(0, _CH // _L, _scale, 0)
                pltpu.sync_copy(rows_x, accs.at[dst_x], add=True)
                if den_on:
                    pltpu.sync_copy(denr_x, dens.at[dst_x], add=True)

            A = (idx_a, dst_a, exb_a, denr_a, rows_a, g_a)
            Bf = (idx_b, dst_b, exb_b, denr_b, rows_b, g_b)

            issue(0, A)
            issue(1, Bf)

            def _pair(kk2, carry):
                ret(A)
                issue(kk2 * 2 + 2, A)
                ret(Bf)
                issue(kk2 * 2 + 3, Bf)
                return carry
            lax.fori_loop(0, (_NCHT - 2) // 2, _pair, 0)
            ret(A)
            ret(Bf)

            plsc.subcore_barrier()
            for z in range(_NZ):
                rz = pl.ds(rbase + z * _ZC, _ZC)
                pltpu.sync_copy(accs.at[rz], fbuf)
                pltpu.sync_copy(fbuf, acc_o.at[c, pi, rz])
                if pi < npass - 1:
                    pltpu.sync_copy(zbuf, accs.at[rz])
            plsc.subcore_barrier()
        for z in range(_NZ):
            rz = pl.ds(rbase + z * _ZC, _ZC)
            pltpu.sync_copy(dens.at[rz], fbuf2)
            pltpu.sync_copy(fbuf2, den_o.at[c, rz])

    return k(hv, ast, adt, src, dst, zrows)


def _stage_c(acc1, den1, h1, as1, ad1, w2, b1r, a_s2, a_d2):
    grid = (_N // _TN,)

    def body(acc_ref, den_ref, h1_ref, s1_ref, d1_ref, w2_ref, b1_ref,
             s2w_ref, d2w_ref, h2_ref, as2_ref, ad2_ref):
        acc = acc_ref[...]
        den = den_ref[...]
        h1 = h1_ref[...]
        exs = jnp.exp(_leaky(s1_ref[...] + d1_ref[...]))
        den4 = jnp.concatenate([den[0, :, 0:2], den[1, :, 0:2]], axis=-1)
        den4 = den4 + exs + 1e-16
        outs = []
        for g in range(_H1):
            ung = jnp.concatenate([acc[g, 0], acc[g, 1]], axis=-1)
            hg = h1[:, g * _HID:(g + 1) * _HID]
            ung = ung + exs[:, g:g + 1] * hg
            og = ung / den4[:, g:g + 1] + b1_ref[0, g * _HID:(g + 1) * _HID][None]
            outs.append(og)
        x2 = jnp.concatenate(outs, axis=-1)
        x2 = jnp.where(x2 > 0.0, x2, jnp.exp(x2) - 1.0)
        h2 = jnp.dot(x2, w2_ref[...], preferred_element_type=jnp.float32)
        h2_ref[...] = h2
        as2_ref[...] = jnp.sum(h2 * s2w_ref[...], axis=-1, keepdims=True)
        ad2_ref[...] = jnp.sum(h2 * d2w_ref[...], axis=-1, keepdims=True)

    full = lambda a: pl.BlockSpec(a.shape, lambda i: (0,) * a.ndim)
    return pl.pallas_call(
        body,
        grid=grid,
        in_specs=[
            pl.BlockSpec((_H1, 2, _TN, 64), lambda i: (0, 0, i, 0)),
            pl.BlockSpec((_NC, _TN, 8), lambda i: (0, i, 0)),
            pl.BlockSpec((_TN, _H1 * _HID), lambda i: (i, 0)),
            pl.BlockSpec((_TN, _H1), lambda i: (i, 0)),
            pl.BlockSpec((_TN, _H1), lambda i: (i, 0)),
            full(w2), full(b1r), full(a_s2), full(a_d2),
        ],
        out_specs=[
            pl.BlockSpec((_TN, _HID), lambda i: (i, 0)),
            pl.BlockSpec((_TN, 1), lambda i: (i, 0)),
            pl.BlockSpec((_TN, 1), lambda i: (i, 0)),
        ],
        out_shape=[
            jax.ShapeDtypeStruct((_N, _HID), jnp.float32),
            jax.ShapeDtypeStruct((_N, 1), jnp.float32),
            jax.ShapeDtypeStruct((_N, 1), jnp.float32),
        ],
    )(acc1, den1, h1, as1, ad1, w2, b1r, a_s2, a_d2)


def _stage_e(acc2, den2, h2, as2, ad2, b2r):
    def body(acc_ref, den_ref, h2_ref, s2_ref, d2_ref, b2_ref, out_ref):
        a2 = acc_ref[...]
        h2 = h2_ref[...]
        exs = jnp.exp(_leaky(s2_ref[...] + d2_ref[...]))
        un = jnp.concatenate([a2[0, :_N], a2[1, :_N]], axis=-1)
        un = un + exs * h2
        den = den_ref[...][0, :_N, 0:1] + exs + 1e-16
        o = un / den + b2_ref[...]
        out_ref[...] = jnp.mean(o.reshape(_B, _NF, _HID), axis=1)

    full = lambda a: pl.BlockSpec(a.shape, lambda: (0,) * a.ndim)
    return pl.pallas_call(
        body,
        in_specs=[
            full(acc2), full(den2), full(h2), full(as2), full(ad2), full(b2r),
        ],
        out_specs=[pl.BlockSpec((_B, _HID), lambda: (0, 0))],
        out_shape=[jax.ShapeDtypeStruct((_B, _HID), jnp.float32)],
    )(acc2, den2, h2, as2, ad2, b2r)[0]


def kernel(field_emb, W_align, b_align, ln_g, ln_b, mask_logits,
           W1, a_src1, a_dst1, b1, W2, a_src2, a_dst2, b2,
           edge_index, batch_idx):
    fe = field_emb.reshape(_N, _FD)
    logits_rows = jnp.tile(mask_logits, _B)[:, None]
    h1, as1, ad1, gate_rows = _stage_a(
        fe, W_align, b_align.reshape(1, -1), ln_g.reshape(1, -1),
        ln_b.reshape(1, -1), logits_rows, W1, a_src1, a_dst1)
    src = edge_index[0]
    dst = edge_index[1]
    zrows = jnp.zeros((_CH, 8), jnp.float32)
    conv1_passes = (
        (0, 0, True, 0, False),
        (1, 0, False, 0, True),
        (2, 1, True, 1, False),
        (3, 1, False, 1, True),
    )
    acc1, den1 = _gat_conv_sc(h1.reshape(_N * 8, 64),
                              as1.T.reshape(_NC, 2, _N),
                              ad1.T.reshape(_NC, 2, _N), src, dst, zrows,
                              8, 4, conv1_passes)
    acc1 = acc1.reshape(_H1, 2, _NP, 64)
    h2, as2, ad2 = _stage_c(acc1, den1, h1, as1, ad1, W2,
                            b1.reshape(1, -1), a_src2, a_dst2)
    as2b = jnp.broadcast_to(as2.reshape(1, 1, _N), (_NC, 2, _N))
    ad2b = jnp.broadcast_to(ad2.reshape(1, 1, _N), (_NC, 2, _N))
    acc2, den2 = _gat_conv_sc(h2.reshape(_N * 2, 64),
                              as2b, ad2b, src, dst, zrows,
                              2, 1, ((0, 0, True, 0, False),))
    acc2 = acc2.reshape(_NC, _NP, 64)
    ge = _stage_e(acc2, den2, h2, as2, ad2, b2.reshape(1, -1))
    gate = gate_rows[:_NF, 0]
    return (ge, gate)


# speedup vs baseline: 1.4491x; 1.2911x over previous
import functools

import jax
import jax.numpy as jnp
from jax import lax
from jax.experimental import pallas as pl
from jax.experimental.pallas import tpu as pltpu
from jax.experimental.pallas import tpu_sc as plsc

_B = 500
_NF = 20
_N = _B * _NF
_E = 160000
_FD = 64
_HID = 128
_H1 = 4

_NC = 2
_NS = 16
_L = 16
_EPT = _E // _NS
_CH = 80
_NCH = _EPT // _CH
_NCHT = 126
_EPTP = _NCHT * _CH
_NP = 10240
_RPT = _NP // _NS
_ZC = 128
_NZ = _RPT // _ZC
_TN = 1000

_SC_PARAMS = pltpu.CompilerParams(needs_layout_passes=False,
                                  use_tc_tiling_on_sc=False)


def _leaky(x):
    return jnp.where(x >= 0.0, x, 0.2 * x)


def _stage_a(fe, w_align, b_align, ln_g, ln_b, logits_rows, w1, a_src1, a_dst1):
    grid = (_N // _TN,)

    def body(fe_ref, wa_ref, ba_ref, lg_ref, lb_ref, ml_ref, w1_ref, s_ref, d_ref,
             h1_ref, as_ref, ad_ref, gate_ref):
        x = jnp.dot(fe_ref[...], wa_ref[...], preferred_element_type=jnp.float32)
        x = x + ba_ref[...]
        mu = jnp.mean(x, axis=-1, keepdims=True)
        xc = x - mu
        var = jnp.mean(xc * xc, axis=-1, keepdims=True)
        xn = xc * lax.rsqrt(var + 1e-5) * lg_ref[...] + lb_ref[...]
        gate = jax.nn.sigmoid(ml_ref[...])
        xg = xn * gate
        h1 = jnp.dot(xg, w1_ref[...], preferred_element_type=jnp.float32)
        h1_ref[...] = h1
        hr = h1.reshape(_TN, _H1, _HID)
        as_ref[...] = jnp.sum(hr * s_ref[...][None], axis=-1)
        ad_ref[...] = jnp.sum(hr * d_ref[...][None], axis=-1)
        gate_ref[...] = gate

    full = lambda a: pl.BlockSpec(a.shape, lambda i: (0,) * a.ndim)
    return pl.pallas_call(
        body,
        grid=grid,
        in_specs=[
            pl.BlockSpec((_TN, _FD), lambda i: (i, 0)),
            full(w_align), full(b_align), full(ln_g), full(ln_b),
            pl.BlockSpec((_TN, 1), lambda i: (i, 0)),
            full(w1), full(a_src1), full(a_dst1),
        ],
        out_specs=[
            pl.BlockSpec((_TN, _H1 * _HID), lambda i: (i, 0)),
            pl.BlockSpec((_TN, _H1), lambda i: (i, 0)),
            pl.BlockSpec((_TN, _H1), lambda i: (i, 0)),
            pl.BlockSpec((_TN, 1), lambda i: (i, 0)),
        ],
        out_shape=[
            jax.ShapeDtypeStruct((_N, _H1 * _HID), jnp.float32),
            jax.ShapeDtypeStruct((_N, _H1), jnp.float32),
            jax.ShapeDtypeStruct((_N, _H1), jnp.float32),
            jax.ShapeDtypeStruct((_N, 1), jnp.float32),
        ],
    )(fe, w_align, b_align, ln_g, ln_b, logits_rows, w1, a_src1, a_dst1)


def _gat_conv_sc(hv, ast, adt, src, dst, zrows, mult, cmul, passes):
    W = 64
    npass = len(passes)
    mesh = plsc.VectorSubcoreMesh(core_axis_name="c", subcore_axis_name="s",
                                  num_cores=_NC, num_subcores=_NS)
    out_type = (
        jax.ShapeDtypeStruct((_NC, npass, _NP, W), jnp.float32),
        jax.ShapeDtypeStruct((_NC, _NP, 8), jnp.float32),
    )
    scratch = [
        pltpu.VMEM((_EPTP,), jnp.int32),
        pltpu.VMEM((_EPTP,), jnp.int32),
        pltpu.VMEM((_N,), jnp.float32),
        pltpu.VMEM((_N,), jnp.float32),
        pltpu.VMEM((_EPTP,), jnp.float32),
        pltpu.VMEM((_CH,), jnp.int32),
        pltpu.VMEM((_CH,), jnp.int32),
        pltpu.VMEM((_CH,), jnp.int32),
        pltpu.VMEM((_CH,), jnp.int32),
        pltpu.VMEM((_CH,), jnp.float32),
        pltpu.VMEM((_CH,), jnp.float32),
        pltpu.VMEM((_CH, 8), jnp.float32),
        pltpu.VMEM((_CH, 8), jnp.float32),
        pltpu.VMEM((_CH, W), jnp.float32),
        pltpu.VMEM((_CH, W), jnp.float32),
        pltpu.VMEM((_ZC, W), jnp.float32),
        pltpu.VMEM((_ZC, 8), jnp.float32),
        pltpu.VMEM((_ZC, W), jnp.float32),
        pltpu.VMEM((_ZC, 8), jnp.float32),
        pltpu.VMEM_SHARED((_NP, W), jnp.float32),
        pltpu.VMEM_SHARED((_NP, 8), jnp.float32),
        pltpu.SemaphoreType.DMA,
        pltpu.SemaphoreType.DMA,
    ]

    @functools.partial(pl.kernel, out_type=out_type, mesh=mesh,
                       compiler_params=_SC_PARAMS, scratch_types=scratch)
    def k(hv_h, as_h, ad_h, src_h, dst_h, zr_h, acc_o, den_o,
          src_l, dst_l, as_tr, ad_tr, exc,
          idx_a, idx_b, dst_a, dst_b, exb_a, exb_b, denr_a, denr_b,
          rows_a, rows_b, zbuf, zbuf2, fbuf, fbuf2, accs, dens, g_a, g_b):
        c = lax.axis_index("c")
        s = lax.axis_index("s")
        ebase = s * _EPT
        rbase = s * _RPT
        pltpu.sync_copy(src_h.at[pl.ds(ebase, _EPT)],
                        src_l.at[pl.ds(0, _EPT)])
        pltpu.sync_copy(dst_h.at[pl.ds(ebase, _EPT)],
                        dst_l.at[pl.ds(0, _EPT)])
        pltpu.sync_copy(as_h.at[c, passes[0][1]], as_tr)
        pltpu.sync_copy(ad_h.at[c, passes[0][1]], ad_tr)

        zv = jnp.zeros((_L,), jnp.float32)
        zvi = jnp.zeros((_L,), jnp.int32)
        iota16 = lax.iota(jnp.int32, _L)

        for j in range((_EPTP - _EPT) // _L):
            src_l[pl.ds(_EPT + j * _L, _L)] = zvi
            dst_l[pl.ds(_EPT + j * _L, _L)] = zvi
            exc[pl.ds(_EPT + j * _L, _L)] = zv

        def _zb(i, carry):
            for f in range(W // _L):
                zbuf[i, pl.ds(f * _L, _L)] = zv
            return carry
        lax.fori_loop(0, _ZC, _zb, 0)
        pltpu.sync_copy(zr_h.at[pl.ds(0, _ZC)], zbuf2)

        def _zdr():
            pltpu.sync_copy(zr_h.at[pl.ds(0, _CH)], denr_a)
            pltpu.sync_copy(zr_h.at[pl.ds(0, _CH)], denr_b)

        for z in range(_NZ):
            pltpu.sync_copy(zbuf, accs.at[pl.ds(rbase + z * _ZC, _ZC)])
            pltpu.sync_copy(zbuf2, dens.at[pl.ds(rbase + z * _ZC, _ZC)])
        plsc.subcore_barrier()

        ept_s = jnp.full((_L,), _EPT, jnp.int32)

        prev_tid = passes[0][1]
        for pi, (o_p, tid, den_on, den_col, from_cache) in enumerate(passes):
            gq = c * cmul + o_p
            if tid != prev_tid:
                pltpu.sync_copy(as_h.at[c, tid], as_tr)
                pltpu.sync_copy(ad_h.at[c, tid], ad_tr)
                prev_tid = tid
            if den_on:
                _zdr()

            def score(ch, idx_x, dst_x, exb_x, denr_x, den_on=den_on,
                      gq=gq, den_col=den_col, from_cache=from_cache):
                off = ch * _CH
                for j in range(_CH // _L):
                    o = pl.ds(off + j * _L, _L)
                    sj = src_l[o]
                    if from_cache:
                        ex = exc[o]
                    else:
                        dj = dst_l[o]
                        av = plsc.load_gather(as_tr, [sj])
                        bv = plsc.load_gather(ad_tr, [dj])
                        ex = jnp.exp(_leaky(av + bv))
                        m = (iota16 + (off + j * _L)) < ept_s
                        ex = jnp.where(m, ex, 0.0)
                        exc[o] = ex
                    oc = pl.ds(j * _L, _L)
                    exb_x[oc] = ex
                    idx_x[oc] = sj * mult + gq
                    dst_x[oc] = dst_l[o]
                    if den_on:
                        plsc.store_scatter(
                            denr_x,
                            [iota16 + j * _L,
                             jnp.full((_L,), den_col, jnp.int32)],
                            ex)

            def issue(ch, buf):
                idx_x, dst_x, exb_x, denr_x, rows_x, g_x = buf
                score(ch, idx_x, dst_x, exb_x, denr_x)
                pltpu.async_copy(hv_h.at[idx_x], rows_x, g_x)

            def ret(buf, den_on=den_on):
                idx_x, dst_x, exb_x, denr_x, rows_x, g_x = buf
                pltpu.make_async_copy(hv_h.at[idx_x], rows_x, g_x).wait()

                def _scale(qq, cc):
                    exv = exb_x[pl.ds(qq * _L, _L)]
                    for lane in range(_L):
                        i = qq * _L + lane
                        exs = exv[lane]
                        for f in range(W // _L):
                            fo = pl.ds(f * _L, _L)
                            rows_x[i, fo] = rows_x[i, fo] * exs
                    return cc
                lax.fori_loop(0, _CH // _L, _scale, 0)
                pltpu.sync_copy(rows_x, accs.at[dst_x], add=True)
                if den_on:
                    pltpu.sync_copy(denr_x, dens.at[dst_x], add=True)

            A = (idx_a, dst_a, exb_a, denr_a, rows_a, g_a)
            Bf = (idx_b, dst_b, exb_b, denr_b, rows_b, g_b)

            issue(0, A)
            issue(1, Bf)

            def _pair(kk2, carry):
                ret(A)
                issue(kk2 * 2 + 2, A)
                ret(Bf)
                issue(kk2 * 2 + 3, Bf)
                return carry
            lax.fori_loop(0, (_NCHT - 2) // 2, _pair, 0)
            ret(A)
            ret(Bf)

            plsc.subcore_barrier()
            for z in range(_NZ):
                rz = pl.ds(rbase + z * _ZC, _ZC)
                pltpu.sync_copy(accs.at[rz], fbuf)
                pltpu.sync_copy(fbuf, acc_o.at[c, pi, rz])
                if pi < npass - 1:
                    pltpu.sync_copy(zbuf, accs.at[rz])
            plsc.subcore_barrier()
        for z in range(_NZ):
            rz = pl.ds(rbase + z * _ZC, _ZC)
            pltpu.sync_copy(dens.at[rz], fbuf2)
            pltpu.sync_copy(fbuf2, den_o.at[c, rz])

    return k(hv, ast, adt, src, dst, zrows)


def _stage_c(acc1, den1, h1, as1, ad1, w2, b1r, a_s2, a_d2):
    grid = (_N // _TN,)

    def body(acc_ref, den_ref, h1_ref, s1_ref, d1_ref, w2_ref, b1_ref,
             s2w_ref, d2w_ref, h2_ref, as2_ref, ad2_ref):
        acc = acc_ref[...]
        den = den_ref[...]
        h1 = h1_ref[...]
        exs = jnp.exp(_leaky(s1_ref[...] + d1_ref[...]))
        den4 = jnp.concatenate([den[0, :, 0:2], den[1, :, 0:2]], axis=-1)
        den4 = den4 + exs + 1e-16
        outs = []
        for g in range(_H1):
            ung = jnp.concatenate([acc[g, 0], acc[g, 1]], axis=-1)
            hg = h1[:, g * _HID:(g + 1) * _HID]
            ung = ung + exs[:, g:g + 1] * hg
            og = ung / den4[:, g:g + 1] + b1_ref[0, g * _HID:(g + 1) * _HID][None]
            outs.append(og)
        x2 = jnp.concatenate(outs, axis=-1)
        x2 = jnp.where(x2 > 0.0, x2, jnp.exp(x2) - 1.0)
        h2 = jnp.dot(x2, w2_ref[...], preferred_element_type=jnp.float32)
        h2_ref[...] = h2
        as2_ref[...] = jnp.sum(h2 * s2w_ref[...], axis=-1, keepdims=True)
        ad2_ref[...] = jnp.sum(h2 * d2w_ref[...], axis=-1, keepdims=True)

    full = lambda a: pl.BlockSpec(a.shape, lambda i: (0,) * a.ndim)
    return pl.pallas_call(
        body,
        grid=grid,
        in_specs=[
            pl.BlockSpec((_H1, 2, _TN, 64), lambda i: (0, 0, i, 0)),
            pl.BlockSpec((_NC, _TN, 8), lambda i: (0, i, 0)),
            pl.BlockSpec((_TN, _H1 * _HID), lambda i: (i, 0)),
            pl.BlockSpec((_TN, _H1), lambda i: (i, 0)),
            pl.BlockSpec((_TN, _H1), lambda i: (i, 0)),
            full(w2), full(b1r), full(a_s2), full(a_d2),
        ],
        out_specs=[
            pl.BlockSpec((_TN, _HID), lambda i: (i, 0)),
            pl.BlockSpec((_TN, 1), lambda i: (i, 0)),
            pl.BlockSpec((_TN, 1), lambda i: (i, 0)),
        ],
        out_shape=[
            jax.ShapeDtypeStruct((_N, _HID), jnp.float32),
            jax.ShapeDtypeStruct((_N, 1), jnp.float32),
            jax.ShapeDtypeStruct((_N, 1), jnp.float32),
        ],
    )(acc1, den1, h1, as1, ad1, w2, b1r, a_s2, a_d2)


def _stage_e(acc2, den2, h2, as2, ad2, b2r):
    def body(acc_ref, den_ref, h2_ref, s2_ref, d2_ref, b2_ref, out_ref):
        a2 = acc_ref[...]
        h2 = h2_ref[...]
        exs = jnp.exp(_leaky(s2_ref[...] + d2_ref[...]))
        un = jnp.concatenate([a2[0, :_N], a2[1, :_N]], axis=-1)
        un = un + exs * h2
        den = den_ref[...][0, :_N, 0:1] + exs + 1e-16
        o = un / den + b2_ref[...]
        out_ref[...] = jnp.mean(o.reshape(_B, _NF, _HID), axis=1)

    full = lambda a: pl.BlockSpec(a.shape, lambda: (0,) * a.ndim)
    return pl.pallas_call(
        body,
        in_specs=[
            full(acc2), full(den2), full(h2), full(as2), full(ad2), full(b2r),
        ],
        out_specs=[pl.BlockSpec((_B, _HID), lambda: (0, 0))],
        out_shape=[jax.ShapeDtypeStruct((_B, _HID), jnp.float32)],
    )(acc2, den2, h2, as2, ad2, b2r)[0]


def kernel(field_emb, W_align, b_align, ln_g, ln_b, mask_logits,
           W1, a_src1, a_dst1, b1, W2, a_src2, a_dst2, b2,
           edge_index, batch_idx):
    fe = field_emb.reshape(_N, _FD)
    logits_rows = jnp.tile(mask_logits, _B)[:, None]
    h1, as1, ad1, gate_rows = _stage_a(
        fe, W_align, b_align.reshape(1, -1), ln_g.reshape(1, -1),
        ln_b.reshape(1, -1), logits_rows, W1, a_src1, a_dst1)
    src = edge_index[0]
    dst = edge_index[1]
    zrows = jnp.zeros((_ZC, 8), jnp.float32)
    conv1_passes = (
        (0, 0, True, 0, False),
        (1, 0, False, 0, True),
        (2, 1, True, 1, False),
        (3, 1, False, 1, True),
    )
    acc1, den1 = _gat_conv_sc(h1.reshape(_N * 8, 64),
                              as1.T.reshape(_NC, 2, _N),
                              ad1.T.reshape(_NC, 2, _N), src, dst, zrows,
                              8, 4, conv1_passes)
    acc1 = acc1.reshape(_H1, 2, _NP, 64)
    h2, as2, ad2 = _stage_c(acc1, den1, h1, as1, ad1, W2,
                            b1.reshape(1, -1), a_src2, a_dst2)
    as2b = jnp.broadcast_to(as2.reshape(1, 1, _N), (_NC, 2, _N))
    ad2b = jnp.broadcast_to(ad2.reshape(1, 1, _N), (_NC, 2, _N))
    acc2, den2 = _gat_conv_sc(h2.reshape(_N * 2, 64),
                              as2b, ad2b, src, dst, zrows,
                              2, 1, ((0, 0, True, 0, False),))
    acc2 = acc2.reshape(_NC, _NP, 64)
    ge = _stage_e(acc2, den2, h2, as2, ad2, b2.reshape(1, -1))
    gate = gate_rows[:_NF, 0]
    return (ge, gate)
